# baseline (speedup 1.0000x reference)
import functools

import jax
import jax.numpy as jnp
from jax import lax
from jax.experimental import pallas as pl
from jax.experimental.pallas import tpu as pltpu
from jax.experimental.pallas import tpu_sc as plsc

EMB = 64
SCALE = 8.0
NC = 2
NS = 16
L = 16
NW = NC * NS
CHUNK = 128
ET = EMB // 8


@functools.lru_cache(maxsize=None)
def _make(n_s):
    mesh = plsc.VectorSubcoreMesh(
        core_axis_name="c", subcore_axis_name="s",
        num_cores=NC, num_subcores=NS)

    def body(tok_hbm, table_hbm, out_hbm, idx_v, rows0, rows1, buf, sem0, sem1):
        wid = lax.axis_index("s") * NC + lax.axis_index("c")
        pltpu.sync_copy(tok_hbm.at[:, wid], idx_v)
        lane = lax.broadcasted_iota(jnp.int32, (L,), 0)
        row_idx = [lane + (j * L) for j in range(CHUNK // L)]

        def fire(s, rows, sem):
            pltpu.async_copy(table_hbm.at[idx_v.at[s]], rows, sem)

        def drain(s, rows, sem):
            pltpu.make_async_copy(table_hbm.at[idx_v.at[s]], rows, sem).wait()

        def emit(s, rows, sem):
            drain(s, rows, sem)

            @pl.loop(0, EMB)
            def _(e):
                col = jnp.full((L,), e, jnp.int32)
                for j in range(CHUNK // L):
                    v = plsc.load_gather(rows, [row_idx[j], col])
                    buf[e, pl.ds(j * L, L)] = v * SCALE

            for et in range(ET):
                pltpu.sync_copy(buf.at[pl.ds(et * 8, 8)],
                                out_hbm.at[s, et, wid])

        fire(0, rows0, sem0)

        @pl.loop(0, n_s - 2, step=2)
        def _(so):
            fire(so + 1, rows1, sem1)
            emit(so, rows0, sem0)
            fire(so + 2, rows0, sem0)
            emit(so + 1, rows1, sem1)

        fire(n_s - 1, rows1, sem1)
        emit(n_s - 2, rows0, sem0)
        emit(n_s - 1, rows1, sem1)

    return pl.kernel(
        body,
        out_type=jax.ShapeDtypeStruct((n_s, ET, NW, 8, CHUNK), jnp.float32),
        mesh=mesh,
        compiler_params=pltpu.CompilerParams(
            use_tc_tiling_on_sc=False, needs_layout_passes=False),
        scratch_types=[
            pltpu.VMEM((n_s, CHUNK), jnp.int32),
            pltpu.VMEM((CHUNK, EMB), jnp.float32),
            pltpu.VMEM((CHUNK, EMB), jnp.float32),
            pltpu.VMEM((EMB, CHUNK), jnp.float32),
            pltpu.SemaphoreType.DMA,
            pltpu.SemaphoreType.DMA,
        ],
    )


def kernel(tokens, table):
    bt, n_s = tokens.shape
    tok = tokens.astype(jnp.int32).T.reshape(n_s, NW, CHUNK)
    out5 = _make(n_s)(tok, table)
    return out5.transpose(2, 4, 0, 1, 3).reshape(bt, n_s, EMB)


# speedup vs baseline: 1.0751x; 1.0751x over previous
import functools

import jax
import jax.numpy as jnp
from jax import lax
from jax.experimental import pallas as pl
from jax.experimental.pallas import tpu as pltpu
from jax.experimental.pallas import tpu_sc as plsc

EMB = 64
SCALE = 8.0
NC = 2
NS = 16
L = 16
NW = NC * NS
CHUNK = 128
ET = EMB // 8


@functools.lru_cache(maxsize=None)
def _make(n_s):
    mesh = plsc.VectorSubcoreMesh(
        core_axis_name="c", subcore_axis_name="s",
        num_cores=NC, num_subcores=NS)

    def body(tok_hbm, table_hbm, out_hbm,
             tok_v, idx_v, rows0, rows1, buf0, buf1,
             g0, g1, o0, o1):
        wid = lax.axis_index("s") * NC + lax.axis_index("c")
        pltpu.sync_copy(tok_hbm.at[pl.ds(wid * CHUNK, CHUNK)], tok_v)

        lane = lax.broadcasted_iota(jnp.int32, (L,), 0)
        row_idx = [lane + (j * L) for j in range(CHUNK // L)]

        @pl.loop(0, n_s)
        def _(s):
            col = jnp.full((L,), s, jnp.int32)
            for j in range(CHUNK // L):
                idx_v[s, pl.ds(j * L, L)] = plsc.load_gather(
                    tok_v, [row_idx[j], col])

        def fire(s, rows, sem):
            pltpu.async_copy(table_hbm.at[idx_v.at[s]], rows, sem)

        def transpose(rows, buf):
            @pl.loop(0, EMB)
            def _(e):
                col = jnp.full((L,), e, jnp.int32)
                for j in range(CHUNK // L):
                    v = plsc.load_gather(rows, [row_idx[j], col])
                    buf[e // 8, e % 8, pl.ds(j * L, L)] = v * SCALE

        def emit(s, rows, gsem, buf, osem, wait_out):
            pltpu.make_async_copy(table_hbm.at[idx_v.at[s]], rows, gsem).wait()
            if wait_out:
                pltpu.make_async_copy(buf, out_hbm.at[s - 2, :, wid], osem).wait()
            transpose(rows, buf)
            pltpu.async_copy(buf, out_hbm.at[s, :, wid], osem)

        fire(0, rows0, g0)
        fire(1, rows1, g1)
        emit(0, rows0, g0, buf0, o0, False)
        fire(2, rows0, g0)
        emit(1, rows1, g1, buf1, o1, False)
        fire(3, rows1, g1)

        @pl.loop(2, n_s - 2, step=2)
        def _(s):
            emit(s, rows0, g0, buf0, o0, True)
            fire(s + 2, rows0, g0)
            emit(s + 1, rows1, g1, buf1, o1, True)
            fire(s + 3, rows1, g1)

        emit(n_s - 2, rows0, g0, buf0, o0, True)
        emit(n_s - 1, rows1, g1, buf1, o1, True)
        pltpu.make_async_copy(buf0, out_hbm.at[n_s - 2, :, wid], o0).wait()
        pltpu.make_async_copy(buf1, out_hbm.at[n_s - 1, :, wid], o1).wait()

    return pl.kernel(
        body,
        out_type=jax.ShapeDtypeStruct((n_s, ET, NW, 8, CHUNK), jnp.float32),
        mesh=mesh,
        compiler_params=pltpu.CompilerParams(
            use_tc_tiling_on_sc=False, needs_layout_passes=False),
        scratch_types=[
            pltpu.VMEM((CHUNK, n_s), jnp.int32),
            pltpu.VMEM((n_s, CHUNK), jnp.int32),
            pltpu.VMEM((CHUNK, EMB), jnp.float32),
            pltpu.VMEM((CHUNK, EMB), jnp.float32),
            pltpu.VMEM((ET, 8, CHUNK), jnp.float32),
            pltpu.VMEM((ET, 8, CHUNK), jnp.float32),
            pltpu.SemaphoreType.DMA,
            pltpu.SemaphoreType.DMA,
            pltpu.SemaphoreType.DMA,
            pltpu.SemaphoreType.DMA,
        ],
    )


def kernel(tokens, table):
    bt, n_s = tokens.shape
    out5 = _make(n_s)(tokens.astype(jnp.int32), table)
    return out5.transpose(2, 4, 0, 1, 3).reshape(bt, n_s, EMB)


# speedup vs baseline: 1.6039x; 1.4920x over previous
import functools

import jax
import jax.numpy as jnp
from jax import lax
from jax.experimental import pallas as pl
from jax.experimental.pallas import tpu as pltpu
from jax.experimental.pallas import tpu_sc as plsc

EMB = 64
SCALE = 8.0
NC = 2
NS = 16
L = 16
NW = NC * NS
CHUNK = 128


@functools.lru_cache(maxsize=None)
def _make(n_s):
    nb = 4 if n_s % 4 == 0 else (2 if n_s % 2 == 0 else 1)
    mesh = plsc.VectorSubcoreMesh(
        core_axis_name="c", subcore_axis_name="s",
        num_cores=NC, num_subcores=NS)

    def body(tok_hbm, table_hbm, out_hbm, tok_v, idx_v, *bufs):
        rows = bufs[:nb]
        gi = bufs[nb:2 * nb]
        wo = bufs[2 * nb:]
        wid = lax.axis_index("s") * NC + lax.axis_index("c")
        pltpu.sync_copy(tok_hbm.at[pl.ds(wid * CHUNK, CHUNK)], tok_v)

        lane = lax.broadcasted_iota(jnp.int32, (L,), 0)
        row_idx = [lane + (j * L) for j in range(CHUNK // L)]

        @pl.loop(0, n_s)
        def _(s):
            col = jnp.full((L,), s, jnp.int32)
            for j in range(CHUNK // L):
                idx_v[s, pl.ds(j * L, L)] = plsc.load_gather(
                    tok_v, [row_idx[j], col])

        def fire_gather(s, b):
            pltpu.async_copy(table_hbm.at[idx_v.at[s]], rows[b], gi[b])

        def drain_gather(s, b):
            pltpu.make_async_copy(
                table_hbm.at[idx_v.at[s]], rows[b], gi[b]).wait()

        def out_ref(s):
            return out_hbm.at[pl.ds(wid * CHUNK, CHUNK), s]

        def fire_out(s, b):
            pltpu.async_copy(rows[b], out_ref(s), wo[b])

        def drain_out(s, b):
            pltpu.make_async_copy(rows[b], out_ref(s), wo[b]).wait()

        def scale(b):
            r = rows[b]

            @pl.loop(0, CHUNK)
            def _(t):
                for j in range(EMB // L):
                    r[t, pl.ds(j * L, L)] = r[t, pl.ds(j * L, L)] * SCALE

        for b in range(nb):
            fire_gather(b, b)

        @pl.loop(0, n_s - nb, step=nb)
        def _(s0):
            for b in range(nb):
                drain_gather(s0 + b, b)
                scale(b)
                fire_out(s0 + b, b)
            for b in range(nb):
                drain_out(s0 + b, b)
                fire_gather(s0 + nb + b, b)

        for b in range(nb):
            drain_gather(n_s - nb + b, b)
            scale(b)
            fire_out(n_s - nb + b, b)
        for b in range(nb):
            drain_out(n_s - nb + b, b)

    return pl.kernel(
        body,
        out_type=jax.ShapeDtypeStruct((NW * CHUNK, n_s, EMB), jnp.float32),
        mesh=mesh,
        compiler_params=pltpu.CompilerParams(
            use_tc_tiling_on_sc=False, needs_layout_passes=False),
        scratch_types=(
            [pltpu.VMEM((CHUNK, n_s), jnp.int32),
             pltpu.VMEM((n_s, CHUNK), jnp.int32)]
            + [pltpu.VMEM((CHUNK, EMB), jnp.float32) for _ in range(nb)]
            + [pltpu.SemaphoreType.DMA for _ in range(2 * nb)]
        ),
    )


def kernel(tokens, table):
    bt, n_s = tokens.shape
    return _make(n_s)(tokens.astype(jnp.int32), table)
